# baseline (speedup 1.0000x reference)
import functools

import jax
import jax.numpy as jnp
from jax import lax
from jax.experimental import pallas as pl
from jax.experimental.pallas import tpu as pltpu
from jax.experimental.pallas import tpu_sc as plsc

_BN_EPS = 1e-5

_NC = 2
_NS = 16
_NW = _NC * _NS
_B = 128
_NBUF = 3


def _proj_body(x_ref, c_ref, w1x_ref, w1c_ref, hp_ref):
    hp_ref[...] = (
        jnp.dot(x_ref[...], w1x_ref[...].T, preferred_element_type=jnp.float32)
        + jnp.dot(c_ref[...], w1c_ref[...].T, preferred_element_type=jnp.float32)
    )


def _project(x, c, W1):
    n = x.shape[0]
    d_hid = W1.shape[0]
    w1x = W1[:, : x.shape[1]]
    w1c = W1[:, x.shape[1] :]
    return pl.pallas_call(
        _proj_body,
        out_shape=jax.ShapeDtypeStruct((n, d_hid), jnp.float32),
    )(x, c, w1x, w1c)


def _sc_body(nb, rows_per_tile, hp_hbm, idx_hbm, zer_hbm, out_hbm,
             idx_v, rows_v, acc_sh, isem, gsem, ssem):
    cid = lax.axis_index("c")
    sid = lax.axis_index("s")
    w = cid * _NS + sid
    base = sid * rows_per_tile
    pltpu.sync_copy(zer_hbm, acc_sh.at[pl.ds(base, rows_per_tile)])
    plsc.subcore_barrier()

    def i_start(b, k):
        pltpu.async_copy(idx_hbm.at[w, b], idx_v.at[k], isem.at[k])

    def i_wait(k):
        pltpu.make_async_copy(
            idx_hbm.at[0, 0], idx_v.at[k], isem.at[k]
        ).wait()

    def g_start(k):
        pltpu.async_copy(hp_hbm.at[idx_v.at[k, 0]], rows_v.at[k], gsem.at[k])

    def g_wait(k):
        pltpu.make_async_copy(
            hp_hbm.at[idx_v.at[0, 0]], rows_v.at[k], gsem.at[k]
        ).wait()

    def s_start(k):
        pltpu.async_copy(
            rows_v.at[k], acc_sh.at[idx_v.at[k, 1]], ssem.at[k], add=True
        )

    def s_wait(k):
        pltpu.make_async_copy(
            rows_v.at[k], acc_sh.at[idx_v.at[0, 1]], ssem.at[k]
        ).wait()

    for k in range(_NBUF):
        i_start(k, k)

    ng = nb // _NBUF

    @pl.loop(0, ng - 1)
    def _grp(g):
        b0 = g * _NBUF
        for k in range(_NBUF):
            i_wait(k)
            g_start(k)
            g_wait(k)
            s_start(k)
            s_wait(k)
            i_start(b0 + _NBUF + k, k)

    for k in range(_NBUF):
        i_wait(k)
        g_start(k)
        g_wait(k)
        s_start(k)
    for k in range(_NBUF):
        s_wait(k)

    plsc.subcore_barrier()
    pltpu.sync_copy(
        acc_sh.at[pl.ds(base, rows_per_tile)],
        out_hbm.at[cid, pl.ds(base, rows_per_tile)],
    )


def _sc_segment_sum(hp, src, dst, n_pad):
    e = src.shape[0]
    d = hp.shape[1]
    per_w = -(-e // _NW)
    per_w_pad = -(-per_w // (_B * _NBUF)) * (_B * _NBUF)
    nb = per_w_pad // _B
    e_pad = per_w_pad * _NW
    rows_per_tile = n_pad // _NS

    src_p = jnp.zeros((e_pad,), jnp.int32).at[:e].set(src.astype(jnp.int32))
    dst_p = jnp.full((e_pad,), n_pad - 1, jnp.int32).at[:e].set(
        dst.astype(jnp.int32)
    )
    idx = jnp.stack(
        [src_p.reshape(_NW, nb, _B), dst_p.reshape(_NW, nb, _B)], axis=2
    )
    zer = jnp.zeros((rows_per_tile, d), jnp.float32)

    mesh = plsc.VectorSubcoreMesh(
        core_axis_name="c", subcore_axis_name="s", num_cores=_NC,
        num_subcores=_NS,
    )
    fn = pl.kernel(
        functools.partial(_sc_body, nb, rows_per_tile),
        out_type=jax.ShapeDtypeStruct((_NC, n_pad, d), jnp.float32),
        mesh=mesh,
        scratch_types=[
            pltpu.VMEM((_NBUF, 2, _B), jnp.int32),
            pltpu.VMEM((_NBUF, _B, d), jnp.float32),
            pltpu.VMEM_SHARED((n_pad, d), jnp.float32),
            pltpu.SemaphoreType.DMA((_NBUF,)),
            pltpu.SemaphoreType.DMA((_NBUF,)),
            pltpu.SemaphoreType.DMA((_NBUF,)),
        ],
    )
    return fn(hp, idx, zer)


def _mlp_body(n, hp_ref, agg_ref, b1_ref, gamma_ref, beta_ref, w2_ref, b2_ref,
              eps_ref, out_ref):
    hp = hp_ref[...]
    y = (
        (1.0 + eps_ref[0, 0]) * hp
        + agg_ref[0, :n, :]
        + agg_ref[1, :n, :]
        + b1_ref[...]
    )
    mu = jnp.mean(y, axis=0, keepdims=True)
    var = jnp.mean(jnp.square(y - mu), axis=0, keepdims=True)
    yhat = (y - mu) * lax.rsqrt(var + _BN_EPS)
    y2 = jnp.maximum(yhat * gamma_ref[...] + beta_ref[...], 0.0)
    out_ref[...] = (
        jnp.dot(y2, w2_ref[...].T, preferred_element_type=jnp.float32)
        + b2_ref[...]
    )


def _mlp(hp, agg, b1, gamma, beta, W2, b2, eps):
    n, d_hid = hp.shape
    emb = W2.shape[0]
    return pl.pallas_call(
        functools.partial(_mlp_body, n),
        out_shape=jax.ShapeDtypeStruct((n, emb), jnp.float32),
        in_specs=[pl.BlockSpec(memory_space=pltpu.VMEM) for _ in range(7)]
        + [pl.BlockSpec(memory_space=pltpu.SMEM)],
        out_specs=pl.BlockSpec(memory_space=pltpu.VMEM),
    )(
        hp,
        agg,
        b1.reshape(1, d_hid),
        gamma.reshape(1, d_hid),
        beta.reshape(1, d_hid),
        W2,
        b2.reshape(1, emb),
        eps.reshape(1, 1),
    )


def kernel(x, c, edge_index, W1, b1, gamma, beta, W2, b2, eps):
    n = x.shape[0]
    n_pad = -(-(n + 1) // (_NS * 8)) * (_NS * 8)
    hp = _project(x, c, W1)
    agg = _sc_segment_sum(hp, edge_index[0], edge_index[1], n_pad)
    return _mlp(hp, agg, b1, gamma, beta, W2, b2, eps)


# speedup vs baseline: 1.9604x; 1.9604x over previous
import functools

import jax
import jax.numpy as jnp
from jax import lax
from jax.experimental import pallas as pl
from jax.experimental.pallas import tpu as pltpu
from jax.experimental.pallas import tpu_sc as plsc

_BN_EPS = 1e-5

_NC = 2
_NS = 16
_NW = _NC * _NS
_B = 128
_NBUF = 2


def _proj_body(x_ref, c_ref, w1x_ref, w1c_ref, hp_ref):
    hp_ref[...] = (
        jnp.dot(x_ref[...], w1x_ref[...].T, preferred_element_type=jnp.float32)
        + jnp.dot(c_ref[...], w1c_ref[...].T, preferred_element_type=jnp.float32)
    )


def _project(x, c, W1):
    n = x.shape[0]
    d_hid = W1.shape[0]
    w1x = W1[:, : x.shape[1]]
    w1c = W1[:, x.shape[1] :]
    return pl.pallas_call(
        _proj_body,
        out_shape=jax.ShapeDtypeStruct((n, d_hid), jnp.float32),
    )(x, c, w1x, w1c)


def _sc_body(nb, rows_per_tile, hp_hbm, idx_hbm, zer_hbm, out_hbm,
             idx_v, rows_v, acc_sh, gsem, ssem):
    cid = lax.axis_index("c")
    sid = lax.axis_index("s")
    w = cid * _NS + sid
    base = sid * rows_per_tile
    pltpu.sync_copy(zer_hbm, acc_sh.at[pl.ds(base, rows_per_tile)])
    plsc.subcore_barrier()

    nbh = nb // 2

    def g_start(b, k):
        pltpu.async_copy(hp_hbm.at[idx_v.at[b, 0]], rows_v.at[k], gsem.at[k])

    def g_wait(k):
        pltpu.make_async_copy(
            hp_hbm.at[idx_v.at[0, 0]], rows_v.at[k], gsem.at[k]
        ).wait()

    def s_start(b, k):
        pltpu.async_copy(
            rows_v.at[k], acc_sh.at[idx_v.at[b, 1]], ssem.at[k], add=True
        )

    def s_wait(k):
        pltpu.make_async_copy(
            rows_v.at[k], acc_sh.at[idx_v.at[0, 1]], ssem.at[k]
        ).wait()

    for half in range(2):
        pltpu.sync_copy(idx_hbm.at[w, pl.ds(half * nbh, nbh)], idx_v)
        for k in range(_NBUF):
            g_start(k, k)
        ng = nbh // _NBUF

        @pl.loop(0, ng - 1)
        def _grp(g):
            b0 = g * _NBUF
            for k in range(_NBUF):
                g_wait(k)
                s_start(b0 + k, k)
                s_wait(k)
                g_start(b0 + _NBUF + k, k)

        b0 = (ng - 1) * _NBUF
        for k in range(_NBUF):
            g_wait(k)
            s_start(b0 + k, k)
        for k in range(_NBUF):
            s_wait(k)

    plsc.subcore_barrier()
    pltpu.sync_copy(
        acc_sh.at[pl.ds(base, rows_per_tile)],
        out_hbm.at[cid, pl.ds(base, rows_per_tile)],
    )


def _sc_segment_sum(hp, src, dst, n_pad):
    e = src.shape[0]
    d = hp.shape[1]
    per_w = -(-e // _NW)
    per_w_pad = -(-per_w // (_B * _NBUF * 2)) * (_B * _NBUF * 2)
    nb = per_w_pad // _B
    e_pad = per_w_pad * _NW
    rows_per_tile = n_pad // _NS

    src_p = jnp.zeros((e_pad,), jnp.int32).at[:e].set(src.astype(jnp.int32))
    dst_p = jnp.full((e_pad,), n_pad - 1, jnp.int32).at[:e].set(
        dst.astype(jnp.int32)
    )
    idx = jnp.stack(
        [src_p.reshape(_NW, nb, _B), dst_p.reshape(_NW, nb, _B)], axis=2
    )
    zer = jnp.zeros((rows_per_tile, d), jnp.float32)

    mesh = plsc.VectorSubcoreMesh(
        core_axis_name="c", subcore_axis_name="s", num_cores=_NC,
        num_subcores=_NS,
    )
    fn = pl.kernel(
        functools.partial(_sc_body, nb, rows_per_tile),
        out_type=jax.ShapeDtypeStruct((_NC, n_pad, d), jnp.float32),
        mesh=mesh,
        scratch_types=[
            pltpu.VMEM((nb // 2, 2, _B), jnp.int32),
            pltpu.VMEM((_NBUF, _B, d), jnp.float32),
            pltpu.VMEM_SHARED((n_pad, d), jnp.float32),
            pltpu.SemaphoreType.DMA((_NBUF,)),
            pltpu.SemaphoreType.DMA((_NBUF,)),
        ],
    )
    return fn(hp, idx, zer)


def _mlp_body(n, hp_ref, agg_ref, b1_ref, gamma_ref, beta_ref, w2_ref, b2_ref,
              eps_ref, out_ref):
    hp = hp_ref[...]
    y = (
        (1.0 + eps_ref[0, 0]) * hp
        + agg_ref[0, :n, :]
        + agg_ref[1, :n, :]
        + b1_ref[...]
    )
    mu = jnp.mean(y, axis=0, keepdims=True)
    var = jnp.mean(jnp.square(y - mu), axis=0, keepdims=True)
    yhat = (y - mu) * lax.rsqrt(var + _BN_EPS)
    y2 = jnp.maximum(yhat * gamma_ref[...] + beta_ref[...], 0.0)
    out_ref[...] = (
        jnp.dot(y2, w2_ref[...].T, preferred_element_type=jnp.float32)
        + b2_ref[...]
    )


def _mlp(hp, agg, b1, gamma, beta, W2, b2, eps):
    n, d_hid = hp.shape
    emb = W2.shape[0]
    return pl.pallas_call(
        functools.partial(_mlp_body, n),
        out_shape=jax.ShapeDtypeStruct((n, emb), jnp.float32),
        in_specs=[pl.BlockSpec(memory_space=pltpu.VMEM) for _ in range(7)]
        + [pl.BlockSpec(memory_space=pltpu.SMEM)],
        out_specs=pl.BlockSpec(memory_space=pltpu.VMEM),
    )(
        hp,
        agg,
        b1.reshape(1, d_hid),
        gamma.reshape(1, d_hid),
        beta.reshape(1, d_hid),
        W2,
        b2.reshape(1, emb),
        eps.reshape(1, 1),
    )


def kernel(x, c, edge_index, W1, b1, gamma, beta, W2, b2, eps):
    n = x.shape[0]
    n_pad = -(-(n + 1) // (_NS * 8)) * (_NS * 8)
    hp = _project(x, c, W1)
    agg = _sc_segment_sum(hp, edge_index[0], edge_index[1], n_pad)
    return _mlp(hp, agg, b1, gamma, beta, W2, b2, eps)


# speedup vs baseline: 1.9612x; 1.0004x over previous
import functools

import jax
import jax.numpy as jnp
from jax import lax
from jax.experimental import pallas as pl
from jax.experimental.pallas import tpu as pltpu
from jax.experimental.pallas import tpu_sc as plsc

_BN_EPS = 1e-5

_NC = 2
_NS = 16
_NW = _NC * _NS
_B = 128
_NBUF = 2


def _proj_body(x_ref, c_ref, w1x_ref, w1c_ref, hp_ref):
    hp_ref[...] = (
        jnp.dot(x_ref[...], w1x_ref[...].T, preferred_element_type=jnp.float32)
        + jnp.dot(c_ref[...], w1c_ref[...].T, preferred_element_type=jnp.float32)
    )


def _project(x, c, W1):
    n = x.shape[0]
    d_hid = W1.shape[0]
    w1x = W1[:, : x.shape[1]]
    w1c = W1[:, x.shape[1] :]
    return pl.pallas_call(
        _proj_body,
        out_shape=jax.ShapeDtypeStruct((n, d_hid), jnp.float32),
    )(x, c, w1x, w1c)


def _sc_body(nb, rows_per_tile, hp_hbm, idx_hbm, zer_hbm, out_hbm,
             idx_v, rows_v, acc_sh, gsem, ssem):
    cid = lax.axis_index("c")
    sid = lax.axis_index("s")
    w = cid * _NS + sid
    base = sid * rows_per_tile
    pltpu.sync_copy(zer_hbm, acc_sh.at[pl.ds(base, rows_per_tile)])
    plsc.subcore_barrier()

    nbh = nb // 2

    def g_start(b, k):
        pltpu.async_copy(hp_hbm.at[idx_v.at[b, 0]], rows_v.at[k], gsem.at[k])

    def g_wait(k):
        pltpu.make_async_copy(
            hp_hbm.at[idx_v.at[0, 0]], rows_v.at[k], gsem.at[k]
        ).wait()

    def s_sync(b, k):
        pltpu.sync_copy(rows_v.at[k], acc_sh.at[idx_v.at[b, 1]], add=True)

    for half in range(2):
        pltpu.sync_copy(idx_hbm.at[w, pl.ds(half * nbh, nbh)], idx_v)
        for k in range(_NBUF):
            g_start(k, k)
        ng = nbh // _NBUF

        @pl.loop(0, ng - 1)
        def _grp(g):
            b0 = g * _NBUF
            for k in range(_NBUF):
                g_wait(k)
                s_sync(b0 + k, k)
                g_start(b0 + _NBUF + k, k)

        b0 = (ng - 1) * _NBUF
        for k in range(_NBUF):
            g_wait(k)
            s_sync(b0 + k, k)

    plsc.subcore_barrier()
    pltpu.sync_copy(
        acc_sh.at[pl.ds(base, rows_per_tile)],
        out_hbm.at[cid, pl.ds(base, rows_per_tile)],
    )


def _sc_segment_sum(hp, src, dst, n_pad):
    e = src.shape[0]
    d = hp.shape[1]
    per_w = -(-e // _NW)
    per_w_pad = -(-per_w // (_B * _NBUF * 2)) * (_B * _NBUF * 2)
    nb = per_w_pad // _B
    e_pad = per_w_pad * _NW
    rows_per_tile = n_pad // _NS

    src_p = jnp.zeros((e_pad,), jnp.int32).at[:e].set(src.astype(jnp.int32))
    dst_p = jnp.full((e_pad,), n_pad - 1, jnp.int32).at[:e].set(
        dst.astype(jnp.int32)
    )
    idx = jnp.stack(
        [src_p.reshape(_NW, nb, _B), dst_p.reshape(_NW, nb, _B)], axis=2
    )
    zer = jnp.zeros((rows_per_tile, d), jnp.float32)

    mesh = plsc.VectorSubcoreMesh(
        core_axis_name="c", subcore_axis_name="s", num_cores=_NC,
        num_subcores=_NS,
    )
    fn = pl.kernel(
        functools.partial(_sc_body, nb, rows_per_tile),
        out_type=jax.ShapeDtypeStruct((_NC, n_pad, d), jnp.float32),
        mesh=mesh,
        scratch_types=[
            pltpu.VMEM((nb // 2, 2, _B), jnp.int32),
            pltpu.VMEM((_NBUF, _B, d), jnp.float32),
            pltpu.VMEM_SHARED((n_pad, d), jnp.float32),
            pltpu.SemaphoreType.DMA((_NBUF,)),
            pltpu.SemaphoreType.DMA((_NBUF,)),
        ],
    )
    return fn(hp, idx, zer)


def _mlp_body(n, hp_ref, agg_ref, b1_ref, gamma_ref, beta_ref, w2_ref, b2_ref,
              eps_ref, out_ref):
    hp = hp_ref[...]
    y = (
        (1.0 + eps_ref[0, 0]) * hp
        + agg_ref[0, :n, :]
        + agg_ref[1, :n, :]
        + b1_ref[...]
    )
    mu = jnp.mean(y, axis=0, keepdims=True)
    var = jnp.mean(jnp.square(y - mu), axis=0, keepdims=True)
    yhat = (y - mu) * lax.rsqrt(var + _BN_EPS)
    y2 = jnp.maximum(yhat * gamma_ref[...] + beta_ref[...], 0.0)
    out_ref[...] = (
        jnp.dot(y2, w2_ref[...].T, preferred_element_type=jnp.float32)
        + b2_ref[...]
    )


def _mlp(hp, agg, b1, gamma, beta, W2, b2, eps):
    n, d_hid = hp.shape
    emb = W2.shape[0]
    return pl.pallas_call(
        functools.partial(_mlp_body, n),
        out_shape=jax.ShapeDtypeStruct((n, emb), jnp.float32),
        in_specs=[pl.BlockSpec(memory_space=pltpu.VMEM) for _ in range(7)]
        + [pl.BlockSpec(memory_space=pltpu.SMEM)],
        out_specs=pl.BlockSpec(memory_space=pltpu.VMEM),
    )(
        hp,
        agg,
        b1.reshape(1, d_hid),
        gamma.reshape(1, d_hid),
        beta.reshape(1, d_hid),
        W2,
        b2.reshape(1, emb),
        eps.reshape(1, 1),
    )


def kernel(x, c, edge_index, W1, b1, gamma, beta, W2, b2, eps):
    n = x.shape[0]
    n_pad = -(-(n + 1) // (_NS * 8)) * (_NS * 8)
    hp = _project(x, c, W1)
    agg = _sc_segment_sum(hp, edge_index[0], edge_index[1], n_pad)
    return _mlp(hp, agg, b1, gamma, beta, W2, b2, eps)


# speedup vs baseline: 2.0485x; 1.0446x over previous
import functools

import jax
import jax.numpy as jnp
from jax import lax
from jax.experimental import pallas as pl
from jax.experimental.pallas import tpu as pltpu
from jax.experimental.pallas import tpu_sc as plsc

_BN_EPS = 1e-5

_NC = 2
_NS = 16
_NW = _NC * _NS
_B = 128
_NBUF = 2


def _proj_body(x_ref, c_ref, w1x_ref, w1c_ref, hp_ref):
    hp_ref[...] = (
        jnp.dot(x_ref[...], w1x_ref[...].T, preferred_element_type=jnp.float32)
        + jnp.dot(c_ref[...], w1c_ref[...].T, preferred_element_type=jnp.float32)
    )


def _project(x, c, W1):
    n = x.shape[0]
    d_hid = W1.shape[0]
    w1x = W1[:, : x.shape[1]]
    w1c = W1[:, x.shape[1] :]
    return pl.pallas_call(
        _proj_body,
        out_shape=jax.ShapeDtypeStruct((n, d_hid), jnp.float32),
    )(x, c, w1x, w1c)


def _sc_body(nb, rows_per_tile, hp_hbm, idx_hbm, zer_hbm, out_hbm,
             idx_v, rows_v, acc_sh, gsem, ssem):
    cid = lax.axis_index("c")
    sid = lax.axis_index("s")
    w = cid * _NS + sid
    base = sid * rows_per_tile
    pltpu.sync_copy(zer_hbm, acc_sh.at[pl.ds(base, rows_per_tile)])
    plsc.subcore_barrier()

    nbh = nb // 2

    def g_start(b, k):
        pltpu.async_copy(hp_hbm.at[idx_v.at[b, 0]], rows_v.at[k], gsem.at[k])

    def g_wait(k):
        pltpu.make_async_copy(
            hp_hbm.at[idx_v.at[0, 0]], rows_v.at[k], gsem.at[k]
        ).wait()

    def s_sync(b, k):
        pltpu.sync_copy(rows_v.at[k], acc_sh.at[idx_v.at[b, 1]], add=True)

    for half in range(2):
        pltpu.sync_copy(idx_hbm.at[w, pl.ds(half * nbh, nbh)], idx_v)
        for k in range(_NBUF):
            g_start(k, k)
        ng = nbh // _NBUF

        @pl.loop(0, ng - 1)
        def _grp(g):
            b0 = g * _NBUF
            for k in range(_NBUF):
                g_wait(k)
                s_sync(b0 + k, k)
                g_start(b0 + _NBUF + k, k)

        b0 = (ng - 1) * _NBUF
        for k in range(_NBUF):
            g_wait(k)
            s_sync(b0 + k, k)

    plsc.subcore_barrier()
    pltpu.sync_copy(
        acc_sh.at[pl.ds(base, rows_per_tile)],
        out_hbm.at[cid, pl.ds(base, rows_per_tile)],
    )


def _sc_segment_sum(hp, src, dst, n_pad):
    e = src.shape[0]
    d = hp.shape[1]
    per_w = -(-e // _NW)
    per_w_pad = -(-per_w // (_B * _NBUF * 2)) * (_B * _NBUF * 2)
    nb = per_w_pad // _B
    e_pad = per_w_pad * _NW
    rows_per_tile = n_pad // _NS

    n = hp.shape[0]
    dspread = n_pad - n
    e_w = _NW * per_w
    src_f = jnp.zeros((e_w,), jnp.int32).at[:e].set(src.astype(jnp.int32))
    dst_f = (n + jnp.arange(e_w, dtype=jnp.int32) % dspread).at[:e].set(
        dst.astype(jnp.int32)
    )
    pad_cnt = per_w_pad - per_w
    src_w = jnp.concatenate(
        [src_f.reshape(_NW, per_w), jnp.zeros((_NW, pad_cnt), jnp.int32)],
        axis=1,
    )
    dst_pad = jnp.broadcast_to(
        n + jnp.arange(pad_cnt, dtype=jnp.int32) % dspread, (_NW, pad_cnt)
    )
    dst_w = jnp.concatenate([dst_f.reshape(_NW, per_w), dst_pad], axis=1)
    idx = jnp.stack(
        [src_w.reshape(_NW, nb, _B), dst_w.reshape(_NW, nb, _B)], axis=2
    )
    zer = jnp.zeros((rows_per_tile, d), jnp.float32)

    mesh = plsc.VectorSubcoreMesh(
        core_axis_name="c", subcore_axis_name="s", num_cores=_NC,
        num_subcores=_NS,
    )
    fn = pl.kernel(
        functools.partial(_sc_body, nb, rows_per_tile),
        out_type=jax.ShapeDtypeStruct((_NC, n_pad, d), jnp.float32),
        mesh=mesh,
        scratch_types=[
            pltpu.VMEM((nb // 2, 2, _B), jnp.int32),
            pltpu.VMEM((_NBUF, _B, d), jnp.float32),
            pltpu.VMEM_SHARED((n_pad, d), jnp.float32),
            pltpu.SemaphoreType.DMA((_NBUF,)),
            pltpu.SemaphoreType.DMA((_NBUF,)),
        ],
    )
    return fn(hp, idx, zer)


def _mlp_body(n, hp_ref, agg_ref, b1_ref, gamma_ref, beta_ref, w2_ref, b2_ref,
              eps_ref, out_ref):
    hp = hp_ref[...]
    y = (
        (1.0 + eps_ref[0, 0]) * hp
        + agg_ref[0, :n, :]
        + agg_ref[1, :n, :]
        + b1_ref[...]
    )
    mu = jnp.mean(y, axis=0, keepdims=True)
    var = jnp.mean(jnp.square(y - mu), axis=0, keepdims=True)
    yhat = (y - mu) * lax.rsqrt(var + _BN_EPS)
    y2 = jnp.maximum(yhat * gamma_ref[...] + beta_ref[...], 0.0)
    out_ref[...] = (
        jnp.dot(y2, w2_ref[...].T, preferred_element_type=jnp.float32)
        + b2_ref[...]
    )


def _mlp(hp, agg, b1, gamma, beta, W2, b2, eps):
    n, d_hid = hp.shape
    emb = W2.shape[0]
    return pl.pallas_call(
        functools.partial(_mlp_body, n),
        out_shape=jax.ShapeDtypeStruct((n, emb), jnp.float32),
        in_specs=[pl.BlockSpec(memory_space=pltpu.VMEM) for _ in range(7)]
        + [pl.BlockSpec(memory_space=pltpu.SMEM)],
        out_specs=pl.BlockSpec(memory_space=pltpu.VMEM),
    )(
        hp,
        agg,
        b1.reshape(1, d_hid),
        gamma.reshape(1, d_hid),
        beta.reshape(1, d_hid),
        W2,
        b2.reshape(1, emb),
        eps.reshape(1, 1),
    )


def kernel(x, c, edge_index, W1, b1, gamma, beta, W2, b2, eps):
    n = x.shape[0]
    n_pad = -(-(n + 1) // (_NS * 8)) * (_NS * 8)
    hp = _project(x, c, W1)
    agg = _sc_segment_sum(hp, edge_index[0], edge_index[1], n_pad)
    return _mlp(hp, agg, b1, gamma, beta, W2, b2, eps)


# speedup vs baseline: 2.0497x; 1.0006x over previous
import functools

import jax
import jax.numpy as jnp
from jax import lax
from jax.experimental import pallas as pl
from jax.experimental.pallas import tpu as pltpu
from jax.experimental.pallas import tpu_sc as plsc

_BN_EPS = 1e-5

_NC = 2
_NS = 16
_NW = _NC * _NS
_B = 128
_NBUF = 2
_NSTAGE = 2


def _proj_body(x_ref, c_ref, w1x_ref, w1c_ref, hp_ref):
    hp_ref[...] = (
        jnp.dot(x_ref[...], w1x_ref[...].T, preferred_element_type=jnp.float32)
        + jnp.dot(c_ref[...], w1c_ref[...].T, preferred_element_type=jnp.float32)
    )


def _project(x, c, W1):
    n = x.shape[0]
    d_hid = W1.shape[0]
    w1x = W1[:, : x.shape[1]]
    w1c = W1[:, x.shape[1] :]
    return pl.pallas_call(
        _proj_body,
        out_shape=jax.ShapeDtypeStruct((n, d_hid), jnp.float32),
    )(x, c, w1x, w1c)


def _sc_body(nb, rows_per_tile, hp_hbm, idx_hbm, zer_hbm, out_hbm,
             idx_v, rows_v, acc_sh, gsem, ssem):
    cid = lax.axis_index("c")
    sid = lax.axis_index("s")
    w = cid * _NS + sid
    base = sid * rows_per_tile
    pltpu.sync_copy(zer_hbm, acc_sh.at[pl.ds(base, rows_per_tile)])
    plsc.subcore_barrier()

    nbh = nb // _NSTAGE

    def g_start(b, k):
        pltpu.async_copy(hp_hbm.at[idx_v.at[b, 0]], rows_v.at[k], gsem.at[k])

    def g_wait(k):
        pltpu.make_async_copy(
            hp_hbm.at[idx_v.at[0, 0]], rows_v.at[k], gsem.at[k]
        ).wait()

    def s_start(b, k):
        pltpu.async_copy(
            rows_v.at[k], acc_sh.at[idx_v.at[b, 1]], ssem.at[k], add=True
        )

    def s_wait(k):
        pltpu.make_async_copy(
            rows_v.at[k], acc_sh.at[idx_v.at[0, 1]], ssem.at[k]
        ).wait()

    for stage in range(_NSTAGE):
        pltpu.sync_copy(idx_hbm.at[w, pl.ds(stage * nbh, nbh)], idx_v)
        for k in range(_NBUF):
            g_start(k, k)
        ng = nbh // _NBUF

        @pl.loop(0, ng - 1)
        def _grp(g):
            b0 = g * _NBUF
            for k in range(_NBUF):
                g_wait(k)
                s_start(b0 + k, k)
                s_wait(k)
                g_start(b0 + _NBUF + k, k)

        b0 = (ng - 1) * _NBUF
        for k in range(_NBUF):
            g_wait(k)
            s_start(b0 + k, k)
        for k in range(_NBUF):
            s_wait(k)

    plsc.subcore_barrier()
    pltpu.sync_copy(
        acc_sh.at[pl.ds(base, rows_per_tile)],
        out_hbm.at[cid, pl.ds(base, rows_per_tile)],
    )


def _sc_segment_sum(hp, src, dst, n_pad):
    e = src.shape[0]
    d = hp.shape[1]
    n = hp.shape[0]
    per_w = -(-e // _NW)
    per_w_pad = -(-per_w // (_B * _NBUF * _NSTAGE)) * (_B * _NBUF * _NSTAGE)
    nb = per_w_pad // _B
    rows_per_tile = n_pad // _NS

    dspread = n_pad - n
    e_w = _NW * per_w
    src_f = jnp.zeros((e_w,), jnp.int32).at[:e].set(src.astype(jnp.int32))
    dst_f = (n + jnp.arange(e_w, dtype=jnp.int32) % dspread).at[:e].set(
        dst.astype(jnp.int32)
    )
    pad_cnt = per_w_pad - per_w
    src_w = jnp.concatenate(
        [src_f.reshape(_NW, per_w), jnp.zeros((_NW, pad_cnt), jnp.int32)],
        axis=1,
    )
    dst_pad = jnp.broadcast_to(
        n + jnp.arange(pad_cnt, dtype=jnp.int32) % dspread, (_NW, pad_cnt)
    )
    dst_w = jnp.concatenate([dst_f.reshape(_NW, per_w), dst_pad], axis=1)
    idx = jnp.stack(
        [src_w.reshape(_NW, nb, _B), dst_w.reshape(_NW, nb, _B)], axis=2
    )
    zer = jnp.zeros((rows_per_tile, d), jnp.float32)

    mesh = plsc.VectorSubcoreMesh(
        core_axis_name="c", subcore_axis_name="s", num_cores=_NC,
        num_subcores=_NS,
    )
    fn = pl.kernel(
        functools.partial(_sc_body, nb, rows_per_tile),
        out_type=jax.ShapeDtypeStruct((_NC, n_pad, d), jnp.float32),
        mesh=mesh,
        scratch_types=[
            pltpu.VMEM((nb // _NSTAGE, 2, _B), jnp.int32),
            pltpu.VMEM((_NBUF, _B, d), jnp.float32),
            pltpu.VMEM_SHARED((n_pad, d), jnp.float32),
            pltpu.SemaphoreType.DMA((_NBUF,)),
            pltpu.SemaphoreType.DMA((_NBUF,)),
        ],
    )
    return fn(hp, idx, zer)


def _mlp_body(n, hp_ref, agg_ref, b1_ref, gamma_ref, beta_ref, w2_ref, b2_ref,
              eps_ref, out_ref):
    hp = hp_ref[...]
    y = (
        (1.0 + eps_ref[0, 0]) * hp
        + agg_ref[0, :n, :]
        + agg_ref[1, :n, :]
        + b1_ref[...]
    )
    mu = jnp.mean(y, axis=0, keepdims=True)
    var = jnp.mean(jnp.square(y - mu), axis=0, keepdims=True)
    yhat = (y - mu) * lax.rsqrt(var + _BN_EPS)
    y2 = jnp.maximum(yhat * gamma_ref[...] + beta_ref[...], 0.0)
    out_ref[...] = (
        jnp.dot(y2, w2_ref[...].T, preferred_element_type=jnp.float32)
        + b2_ref[...]
    )


def _mlp(hp, agg, b1, gamma, beta, W2, b2, eps):
    n, d_hid = hp.shape
    emb = W2.shape[0]
    return pl.pallas_call(
        functools.partial(_mlp_body, n),
        out_shape=jax.ShapeDtypeStruct((n, emb), jnp.float32),
        in_specs=[pl.BlockSpec(memory_space=pltpu.VMEM) for _ in range(7)]
        + [pl.BlockSpec(memory_space=pltpu.SMEM)],
        out_specs=pl.BlockSpec(memory_space=pltpu.VMEM),
    )(
        hp,
        agg,
        b1.reshape(1, d_hid),
        gamma.reshape(1, d_hid),
        beta.reshape(1, d_hid),
        W2,
        b2.reshape(1, emb),
        eps.reshape(1, 1),
    )


def kernel(x, c, edge_index, W1, b1, gamma, beta, W2, b2, eps):
    n = x.shape[0]
    n_pad = -(-(n + 1) // (_NS * 8)) * (_NS * 8)
    hp = _project(x, c, W1)
    agg = _sc_segment_sum(hp, edge_index[0], edge_index[1], n_pad)
    return _mlp(hp, agg, b1, gamma, beta, W2, b2, eps)
